# Initial kernel scaffold; baseline (speedup 1.0000x reference)
import jax
import jax.numpy as jnp
from jax import lax
from jax.experimental import pallas as pl
from jax.experimental.pallas import tpu as pltpu
from jax.experimental.pallas import tpu_sc as plsc

N = 10000
E = 320000
D = 128

CH = 125
NCH = E // CH
NSUB = 16
NCORE = 2
CPW = NCH // (NCORE * NSUB)
ROWS_PER_SUB = N // NSUB

BE = 2560
BN_ = 2000


def _norm(x):
    return jnp.maximum(jnp.sqrt(jnp.sum(x * x, axis=-1, keepdims=True)), 1e-15)


def _hyp(x):
    n = _norm(x)
    e = jnp.tanh(n) * x / n
    ne = _norm(e)
    maxn = 1.0 - 1e-05
    e = jnp.where(ne > maxn, e / ne * maxn, e)
    n3 = _norm(e)
    atanh = 0.5 * (jnp.log1p(n3) - jnp.log1p(-n3))
    return atanh / n3 * e


def _elu(x):
    return jnp.where(x > 0, x, jnp.exp(jnp.minimum(x, 0.0)) - 1.0)


def _node_body(x_ref, o_ref):
    o_ref[...] = _hyp(x_ref[...])


def _node_transform(features):
    return pl.pallas_call(
        _node_body,
        grid=(N // BN_,),
        in_specs=[pl.BlockSpec((BN_, D), lambda i: (i, 0))],
        out_specs=pl.BlockSpec((BN_, D), lambda i: (i, 0)),
        out_shape=jax.ShapeDtypeStruct((N, D), jnp.float32),
    )(features)


def _gather_body(feats_hbm, ei0_hbm, ei1_hbm, srcg_hbm, dstg_hbm,
                 idx0_v, idx1_v, rows0_v, rows1_v, sem0, sem1):
    c = lax.axis_index("c")
    s = lax.axis_index("s")
    wid = c * NSUB + s

    def body(j, carry):
        ch = wid * CPW + j
        pltpu.sync_copy(ei0_hbm.at[ch], idx0_v)
        pltpu.sync_copy(ei1_hbm.at[ch], idx1_v)
        cp0 = pltpu.async_copy(feats_hbm.at[idx0_v], rows0_v, sem0)
        cp1 = pltpu.async_copy(feats_hbm.at[idx1_v], rows1_v, sem1)
        cp0.wait()
        cp1.wait()
        pltpu.sync_copy(rows0_v, srcg_hbm.at[pl.ds(ch * CH, CH)])
        pltpu.sync_copy(rows1_v, dstg_hbm.at[pl.ds(ch * CH, CH)])
        return carry

    lax.fori_loop(0, CPW, body, 0)


def _gather(feats, ei0, ei1):
    f = pl.kernel(
        _gather_body,
        out_type=(jax.ShapeDtypeStruct((E, D), jnp.float32),
                  jax.ShapeDtypeStruct((E, D), jnp.float32)),
        mesh=plsc.VectorSubcoreMesh(core_axis_name="c", subcore_axis_name="s"),
        scratch_types=[
            pltpu.VMEM((CH,), jnp.int32),
            pltpu.VMEM((CH,), jnp.int32),
            pltpu.VMEM((CH, D), jnp.float32),
            pltpu.VMEM((CH, D), jnp.float32),
            pltpu.SemaphoreType.DMA,
            pltpu.SemaphoreType.DMA,
        ],
    )
    return f(feats, ei0, ei1)


def _edge_body(src_ref, dst_ref, w1a_ref, w1b_ref, b1_ref, lng_ref, lnb_ref,
               w2_ref, b2_ref, h2_ref, s1_ref, s2_ref):
    src = src_ref[...]
    dst = dst_ref[...]
    multi = jnp.sum(src * dst, axis=-1, keepdims=True)
    dd = src - dst
    dist = jnp.sqrt(jnp.sum(dd * dd, axis=-1, keepdims=True))
    z = 2.0 * dist + 2.0 * (dist * dist * dist / 3.0 + multi * dist * dist)
    coef = 1.0 - jax.nn.sigmoid(z)
    h = (jnp.dot((1.0 + coef) * src, w1a_ref[...],
                 preferred_element_type=jnp.float32)
         + jnp.dot(dst, w1b_ref[...], preferred_element_type=jnp.float32)
         + b1_ref[...])
    h = _elu(h)
    mu = jnp.mean(h, axis=-1, keepdims=True)
    hc = h - mu
    var = jnp.mean(hc * hc, axis=-1, keepdims=True)
    h = hc / jnp.sqrt(var + 1e-5) * lng_ref[...] + lnb_ref[...]
    h = jnp.dot(h, w2_ref[...], preferred_element_type=jnp.float32) + b2_ref[...]
    h = _elu(h)
    h2_ref[...] = h

    @pl.when(pl.program_id(0) == 0)
    def _():
        s1_ref[...] = jnp.zeros_like(s1_ref)
        s2_ref[...] = jnp.zeros_like(s2_ref)

    s1_ref[...] += jnp.sum(h, axis=0, keepdims=True)
    s2_ref[...] += jnp.sum(h * h, axis=0, keepdims=True)


def _edge_mlp(srcg, dstg, w1aT, w1bT, b1, ln_g, ln_b, w2T, b2):
    full = pl.BlockSpec((D, D), lambda i: (0, 0))
    vec = pl.BlockSpec((1, D), lambda i: (0, 0))
    return pl.pallas_call(
        _edge_body,
        grid=(E // BE,),
        in_specs=[
            pl.BlockSpec((BE, D), lambda i: (i, 0)),
            pl.BlockSpec((BE, D), lambda i: (i, 0)),
            full, full, vec, vec, vec, full, vec,
        ],
        out_specs=[
            pl.BlockSpec((BE, D), lambda i: (i, 0)),
            vec, vec,
        ],
        out_shape=[
            jax.ShapeDtypeStruct((E, D), jnp.float32),
            jax.ShapeDtypeStruct((1, D), jnp.float32),
            jax.ShapeDtypeStruct((1, D), jnp.float32),
        ],
    )(srcg, dstg, w1aT, w1bT, b1, ln_g, ln_b, w2T, b2)


def _scatter_body(h2_hbm, ei1_hbm, zs_hbm, zd_hbm, ones_hbm, sp_hbm, dp_hbm,
                  idx_v, h2_v, ones_v, s_sh, d_sh):
    c = lax.axis_index("c")
    s = lax.axis_index("s")
    r0 = s * ROWS_PER_SUB
    pltpu.sync_copy(zs_hbm.at[pl.ds(r0, ROWS_PER_SUB)],
                    s_sh.at[pl.ds(r0, ROWS_PER_SUB)])
    pltpu.sync_copy(zd_hbm.at[pl.ds(r0, ROWS_PER_SUB)],
                    d_sh.at[pl.ds(r0, ROWS_PER_SUB)])
    pltpu.sync_copy(ones_hbm, ones_v)
    plsc.subcore_barrier()

    def body(j, carry):
        ch = c * (NSUB * CPW) + s * CPW + j
        pltpu.sync_copy(ei1_hbm.at[ch], idx_v)
        pltpu.sync_copy(h2_hbm.at[pl.ds(ch * CH, CH)], h2_v)
        pltpu.sync_copy(h2_v, s_sh.at[idx_v], add=True)
        pltpu.sync_copy(ones_v, d_sh.at[idx_v], add=True)
        return carry

    lax.fori_loop(0, CPW, body, 0)
    plsc.subcore_barrier()
    pltpu.sync_copy(s_sh.at[pl.ds(r0, ROWS_PER_SUB)],
                    sp_hbm.at[c, pl.ds(r0, ROWS_PER_SUB)])
    pltpu.sync_copy(d_sh.at[pl.ds(r0, ROWS_PER_SUB)],
                    dp_hbm.at[c, pl.ds(r0, ROWS_PER_SUB)])


def _scatter(h2, ei1, zs, zd, ones16):
    f = pl.kernel(
        _scatter_body,
        out_type=(jax.ShapeDtypeStruct((NCORE, N, D), jnp.float32),
                  jax.ShapeDtypeStruct((NCORE, N, 16), jnp.float32)),
        mesh=plsc.VectorSubcoreMesh(core_axis_name="c", subcore_axis_name="s"),
        scratch_types=[
            pltpu.VMEM((CH,), jnp.int32),
            pltpu.VMEM((CH, D), jnp.float32),
            pltpu.VMEM((CH, 16), jnp.float32),
            pltpu.VMEM_SHARED((N, D), jnp.float32),
            pltpu.VMEM_SHARED((N, 16), jnp.float32),
        ],
    )
    return f(h2, ei1, zs, zd, ones16)


def _final_body(sp_ref, deg_ref, s1_ref, s2_ref, bng_ref, bnb_ref, wo_ref,
                bo_ref, feats_ref, o_ref):
    s_sum = sp_ref[0] + sp_ref[1]
    m = s1_ref[...] / float(E)
    v = s2_ref[...] / float(E) - m * m
    a = bng_ref[...] / jnp.sqrt(v + 1e-5)
    cv = bnb_ref[...] - m * a
    kv = jnp.dot(cv, wo_ref[...], preferred_element_type=jnp.float32)
    out = (jnp.dot(s_sum * a, wo_ref[...], preferred_element_type=jnp.float32)
           + deg_ref[...] * kv + bo_ref[...])
    out = _hyp(out)
    out = 1.0507009873554805 * jnp.where(
        out > 0, out, 1.6732632423543772 * (jnp.exp(jnp.minimum(out, 0.0)) - 1.0))
    o_ref[...] = out + feats_ref[...]


def _final(sp, deg, s1, s2, bn_g, bn_b, woT, bo, feats):
    vec = pl.BlockSpec((1, D), lambda i: (0, 0))
    return pl.pallas_call(
        _final_body,
        grid=(N // BN_,),
        in_specs=[
            pl.BlockSpec((NCORE, BN_, D), lambda i: (0, i, 0)),
            pl.BlockSpec((BN_, 1), lambda i: (i, 0)),
            vec, vec, vec, vec,
            pl.BlockSpec((D, D), lambda i: (0, 0)),
            vec,
            pl.BlockSpec((BN_, D), lambda i: (i, 0)),
        ],
        out_specs=pl.BlockSpec((BN_, D), lambda i: (i, 0)),
        out_shape=jax.ShapeDtypeStruct((N, D), jnp.float32),
    )(sp, deg, s1, s2, bn_g, bn_b, woT, bo, feats)


def kernel(features, edge_index, c, W1, b1, ln_g, ln_b, W2, b2, bn_g, bn_b,
           Wo, bo):
    del c
    f32 = jnp.float32

    feats = _node_transform(features)

    ei0 = edge_index[0].reshape(NCH, CH)
    ei1 = edge_index[1].reshape(NCH, CH)
    srcg, dstg = _gather(feats, ei0, ei1)

    h2, s1, s2 = _edge_mlp(
        srcg, dstg,
        W1[:, :D].T, W1[:, D:].T, b1[None], ln_g[None], ln_b[None],
        W2.T, b2[None])

    zs = jnp.zeros((N, D), f32)
    zd = jnp.zeros((N, 16), f32)
    ones16 = jnp.concatenate(
        [jnp.ones((CH, 1), f32), jnp.zeros((CH, 15), f32)], axis=1)
    sp, dp = _scatter(h2, ei1, zs, zd, ones16)

    deg = (dp[0, :, 0] + dp[1, :, 0]).reshape(N, 1)
    return _final(sp, deg, s1, s2, bn_g[None], bn_b[None], Wo.T, bo[None],
                  feats)


# speedup vs baseline: 3.0491x; 3.0491x over previous
import jax
import jax.numpy as jnp
from jax import lax
from jax.experimental import pallas as pl
from jax.experimental.pallas import tpu as pltpu
from jax.experimental.pallas import tpu_sc as plsc

N = 10000
E = 320000
D = 128

CH = 128
NCH = E // CH
NSUB = 16
NCORE = 2
NW = NCORE * NSUB
NPAD = 10240
ROWS_PER_SUB = NPAD // NSUB

BE = 2560
BN_ = 2000


def _norm(x):
    return jnp.maximum(jnp.sqrt(jnp.sum(x * x, axis=-1, keepdims=True)), 1e-15)


def _hyp(x):
    n = _norm(x)
    e = jnp.tanh(n) * x / n
    ne = _norm(e)
    maxn = 1.0 - 1e-05
    e = jnp.where(ne > maxn, e / ne * maxn, e)
    n3 = _norm(e)
    atanh = 0.5 * (jnp.log1p(n3) - jnp.log1p(-n3))
    return atanh / n3 * e


def _elu(x):
    return jnp.where(x > 0, x, jnp.exp(jnp.minimum(x, 0.0)) - 1.0)


def _node_body(x_ref, o_ref):
    o_ref[...] = _hyp(x_ref[...])


def _node_transform(features):
    return pl.pallas_call(
        _node_body,
        grid=(N // BN_,),
        in_specs=[pl.BlockSpec((BN_, D), lambda i: (i, 0))],
        out_specs=pl.BlockSpec((BN_, D), lambda i: (i, 0)),
        out_shape=jax.ShapeDtypeStruct((N, D), jnp.float32),
    )(features)


def _gather_body(feats_hbm, ei0_hbm, ei1_hbm, srcg_hbm, dstg_hbm,
                 idx0_v, idx1_v, rows0_v, rows1_v, sem0, sem1):
    c = lax.axis_index("c")
    s = lax.axis_index("s")
    wid = c * NSUB + s

    def body(j, carry):
        ch = wid + NW * j

        @pl.when(ch < NCH)
        def _():
            base = ch * CH
            pltpu.sync_copy(ei0_hbm.at[pl.ds(base, CH)], idx0_v)
            pltpu.sync_copy(ei1_hbm.at[pl.ds(base, CH)], idx1_v)
            cp0 = pltpu.async_copy(feats_hbm.at[idx0_v], rows0_v, sem0)
            cp1 = pltpu.async_copy(feats_hbm.at[idx1_v], rows1_v, sem1)
            cp0.wait()
            cp1.wait()
            pltpu.sync_copy(rows0_v, srcg_hbm.at[pl.ds(base, CH)])
            pltpu.sync_copy(rows1_v, dstg_hbm.at[pl.ds(base, CH)])

        return carry

    lax.fori_loop(0, (NCH + NW - 1) // NW, body, 0)


def _gather(feats, ei0, ei1):
    f = pl.kernel(
        _gather_body,
        out_type=(jax.ShapeDtypeStruct((E, D), jnp.float32),
                  jax.ShapeDtypeStruct((E, D), jnp.float32)),
        mesh=plsc.VectorSubcoreMesh(core_axis_name="c", subcore_axis_name="s"),
        scratch_types=[
            pltpu.VMEM((CH,), jnp.int32),
            pltpu.VMEM((CH,), jnp.int32),
            pltpu.VMEM((CH, D), jnp.float32),
            pltpu.VMEM((CH, D), jnp.float32),
            pltpu.SemaphoreType.DMA,
            pltpu.SemaphoreType.DMA,
        ],
    )
    return f(feats, ei0, ei1)


def _edge_body(src_ref, dst_ref, w1a_ref, w1b_ref, b1_ref, lng_ref, lnb_ref,
               w2_ref, b2_ref, h2_ref, s1_ref, s2_ref):
    src = src_ref[...]
    dst = dst_ref[...]
    multi = jnp.sum(src * dst, axis=-1, keepdims=True)
    dd = src - dst
    dist = jnp.sqrt(jnp.sum(dd * dd, axis=-1, keepdims=True))
    z = 2.0 * dist + 2.0 * (dist * dist * dist / 3.0 + multi * dist * dist)
    coef = 1.0 - jax.nn.sigmoid(z)
    h = (jnp.dot((1.0 + coef) * src, w1a_ref[...],
                 preferred_element_type=jnp.float32)
         + jnp.dot(dst, w1b_ref[...], preferred_element_type=jnp.float32)
         + b1_ref[...])
    h = _elu(h)
    mu = jnp.mean(h, axis=-1, keepdims=True)
    hc = h - mu
    var = jnp.mean(hc * hc, axis=-1, keepdims=True)
    h = hc / jnp.sqrt(var + 1e-5) * lng_ref[...] + lnb_ref[...]
    h = jnp.dot(h, w2_ref[...], preferred_element_type=jnp.float32) + b2_ref[...]
    h = _elu(h)
    h2_ref[...] = h

    @pl.when(pl.program_id(0) == 0)
    def _():
        s1_ref[...] = jnp.zeros_like(s1_ref)
        s2_ref[...] = jnp.zeros_like(s2_ref)

    s1_ref[...] += jnp.sum(h, axis=0, keepdims=True)
    s2_ref[...] += jnp.sum(h * h, axis=0, keepdims=True)


def _edge_mlp(srcg, dstg, w1aT, w1bT, b1, ln_g, ln_b, w2T, b2):
    full = pl.BlockSpec((D, D), lambda i: (0, 0))
    vec = pl.BlockSpec((1, D), lambda i: (0, 0))
    return pl.pallas_call(
        _edge_body,
        grid=(E // BE,),
        in_specs=[
            pl.BlockSpec((BE, D), lambda i: (i, 0)),
            pl.BlockSpec((BE, D), lambda i: (i, 0)),
            full, full, vec, vec, vec, full, vec,
        ],
        out_specs=[
            pl.BlockSpec((BE, D), lambda i: (i, 0)),
            vec, vec,
        ],
        out_shape=[
            jax.ShapeDtypeStruct((E, D), jnp.float32),
            jax.ShapeDtypeStruct((1, D), jnp.float32),
            jax.ShapeDtypeStruct((1, D), jnp.float32),
        ],
    )(srcg, dstg, w1aT, w1bT, b1, ln_g, ln_b, w2T, b2)


def _scatter_body(h2_hbm, ei1_hbm, zs_hbm, ones_hbm, sp_hbm, dg_hbm,
                  idx_v, h2_v, ones_v, s_sh):
    c = lax.axis_index("c")
    s = lax.axis_index("s")
    wid = c * NSUB + s
    r0 = s * ROWS_PER_SUB

    @pl.when(s == 0)
    def _():
        pltpu.sync_copy(zs_hbm, s_sh)

    pltpu.sync_copy(ones_hbm, ones_v)
    plsc.subcore_barrier()

    nloops = (NCH + NW - 1) // NW

    def body(j, carry):
        ch = wid + NW * j

        @pl.when(ch < NCH)
        def _():
            base = ch * CH
            pltpu.sync_copy(ei1_hbm.at[pl.ds(base, CH)], idx_v)
            pltpu.sync_copy(h2_hbm.at[pl.ds(base, CH)], h2_v)
            pltpu.sync_copy(h2_v, s_sh.at[idx_v], add=True)

        return carry

    lax.fori_loop(0, nloops, body, 0)
    plsc.subcore_barrier()
    pltpu.sync_copy(s_sh.at[pl.ds(r0, ROWS_PER_SUB)],
                    sp_hbm.at[pl.ds(c * NPAD + r0, ROWS_PER_SUB)])
    plsc.subcore_barrier()

    @pl.when(s == 0)
    def _():
        pltpu.sync_copy(zs_hbm, s_sh)

    plsc.subcore_barrier()

    def dbody(j, carry):
        ch = wid + NW * j

        @pl.when(ch < NCH)
        def _():
            base = ch * CH
            pltpu.sync_copy(ei1_hbm.at[pl.ds(base, CH)], idx_v)
            pltpu.sync_copy(ones_v, s_sh.at[idx_v], add=True)

        return carry

    lax.fori_loop(0, nloops, dbody, 0)
    plsc.subcore_barrier()
    pltpu.sync_copy(s_sh.at[pl.ds(r0, ROWS_PER_SUB)],
                    dg_hbm.at[pl.ds(c * NPAD + r0, ROWS_PER_SUB)])


def _scatter(h2, ei1, zs, ones):
    f = pl.kernel(
        _scatter_body,
        out_type=(jax.ShapeDtypeStruct((NCORE * NPAD, D), jnp.float32),
                  jax.ShapeDtypeStruct((NCORE * NPAD, D), jnp.float32)),
        mesh=plsc.VectorSubcoreMesh(core_axis_name="c", subcore_axis_name="s"),
        scratch_types=[
            pltpu.VMEM((CH,), jnp.int32),
            pltpu.VMEM((CH, D), jnp.float32),
            pltpu.VMEM((CH, D), jnp.float32),
            pltpu.VMEM_SHARED((NPAD, D), jnp.float32),
        ],
    )
    return f(h2, ei1, zs, ones)


def _final_body(sp_ref, dg_ref, s1_ref, s2_ref, bng_ref, bnb_ref, wo_ref,
                bo_ref, feats_ref, o_ref):
    s_sum = sp_ref[0] + sp_ref[1]
    d_sum = dg_ref[0] + dg_ref[1]
    m = s1_ref[...] / float(E)
    v = s2_ref[...] / float(E) - m * m
    a = bng_ref[...] / jnp.sqrt(v + 1e-5)
    cv = bnb_ref[...] - m * a
    kv = jnp.dot(cv, wo_ref[...], preferred_element_type=jnp.float32)
    out = (jnp.dot(s_sum * a, wo_ref[...], preferred_element_type=jnp.float32)
           + d_sum * kv + bo_ref[...])
    out = _hyp(out)
    out = 1.0507009873554805 * jnp.where(
        out > 0, out, 1.6732632423543772 * (jnp.exp(jnp.minimum(out, 0.0)) - 1.0))
    o_ref[...] = out + feats_ref[...]


def _final(sp, dg, s1, s2, bn_g, bn_b, woT, bo, feats):
    vec = pl.BlockSpec((1, D), lambda i: (0, 0))
    return pl.pallas_call(
        _final_body,
        grid=(N // BN_,),
        in_specs=[
            pl.BlockSpec((NCORE, BN_, D), lambda i: (0, i, 0)),
            pl.BlockSpec((NCORE, BN_, D), lambda i: (0, i, 0)),
            vec, vec, vec, vec,
            pl.BlockSpec((D, D), lambda i: (0, 0)),
            vec,
            pl.BlockSpec((BN_, D), lambda i: (i, 0)),
        ],
        out_specs=pl.BlockSpec((BN_, D), lambda i: (i, 0)),
        out_shape=jax.ShapeDtypeStruct((N, D), jnp.float32),
    )(sp, dg, s1, s2, bn_g, bn_b, woT, bo, feats)


def kernel(features, edge_index, c, W1, b1, ln_g, ln_b, W2, b2, bn_g, bn_b,
           Wo, bo):
    del c
    f32 = jnp.float32

    feats = _node_transform(features)

    ei0 = edge_index[0]
    ei1 = edge_index[1]
    srcg, dstg = _gather(feats, ei0, ei1)

    h2, s1, s2 = _edge_mlp(
        srcg, dstg,
        W1[:, :D].T, W1[:, D:].T, b1[None], ln_g[None], ln_b[None],
        W2.T, b2[None])

    zs = jnp.zeros((NPAD, D), f32)
    ones = jnp.ones((CH, D), f32)
    sp, dg = _scatter(h2, ei1, zs, ones)
    sp = sp.reshape(NCORE, NPAD, D)[:, :N]
    dg = dg.reshape(NCORE, NPAD, D)[:, :N]
    return _final(sp, dg, s1, s2, bn_g[None], bn_b[None], Wo.T,
                  bo[None], feats)


# speedup vs baseline: 3.7850x; 1.2414x over previous
import jax
import jax.numpy as jnp
from jax import lax
from jax.experimental import pallas as pl
from jax.experimental.pallas import tpu as pltpu
from jax.experimental.pallas import tpu_sc as plsc

N = 10000
E = 320000
D = 128

CH = 128
NCH = E // CH
NSUB = 16
NCORE = 2
NW = NCORE * NSUB
NPAD = 10240
ROWS_PER_SUB = NPAD // NSUB

BE = 2560
BN_ = 2000


def _norm(x):
    return jnp.maximum(jnp.sqrt(jnp.sum(x * x, axis=-1, keepdims=True)), 1e-15)


def _hyp(x):
    n = _norm(x)
    e = jnp.tanh(n) * x / n
    ne = _norm(e)
    maxn = 1.0 - 1e-05
    e = jnp.where(ne > maxn, e / ne * maxn, e)
    n3 = _norm(e)
    atanh = 0.5 * (jnp.log1p(n3) - jnp.log1p(-n3))
    return atanh / n3 * e


def _elu(x):
    return jnp.where(x > 0, x, jnp.exp(jnp.minimum(x, 0.0)) - 1.0)


def _node_body(x_ref, o_ref):
    o_ref[...] = _hyp(x_ref[...])


def _node_transform(features):
    return pl.pallas_call(
        _node_body,
        grid=(N // BN_,),
        in_specs=[pl.BlockSpec((BN_, D), lambda i: (i, 0))],
        out_specs=pl.BlockSpec((BN_, D), lambda i: (i, 0)),
        out_shape=jax.ShapeDtypeStruct((N, D), jnp.float32),
    )(features)


NFULL = NCH // NW


def _gather_body(feats_hbm, ei0_hbm, ei1_hbm, srcg_hbm, dstg_hbm,
                 idx0a, idx1a, idx0b, idx1b, r0a, r1a, r0b, r1b,
                 semi_a, semi_b, semg_a, semg_b, semw_a, semw_b):
    c = lax.axis_index("c")
    s = lax.axis_index("s")
    wid = c * NSUB + s
    idx = ((idx0a, idx1a), (idx0b, idx1b))
    rows = ((r0a, r1a), (r0b, r1b))
    semi = (semi_a, semi_b)
    semg = (semg_a, semg_b)
    semw = (semw_a, semw_b)

    def base_of(j):
        return (wid + NW * j) * CH

    for b in (0, 1):
        pb = base_of(b)
        pltpu.async_copy(ei0_hbm.at[pl.ds(pb, CH)], idx[b][0], semi[b])
        pltpu.async_copy(ei1_hbm.at[pl.ds(pb, CH)], idx[b][1], semi[b])

    def outer(g, carry):
        for b in (0, 1):
            j = g * 2 + b
            base = base_of(j)

            @pl.when(j >= 2)
            def _():
                pv = base_of(j - 2)
                pltpu.make_async_copy(
                    rows[b][0], srcg_hbm.at[pl.ds(pv, CH)], semw[b]).wait()
                pltpu.make_async_copy(
                    rows[b][1], dstg_hbm.at[pl.ds(pv, CH)], semw[b]).wait()

            pltpu.make_async_copy(
                ei0_hbm.at[pl.ds(base, CH)], idx[b][0], semi[b]).wait()
            pltpu.make_async_copy(
                ei1_hbm.at[pl.ds(base, CH)], idx[b][1], semi[b]).wait()
            cp0 = pltpu.async_copy(feats_hbm.at[idx[b][0]], rows[b][0],
                                   semg[b])
            cp1 = pltpu.async_copy(feats_hbm.at[idx[b][1]], rows[b][1],
                                   semg[b])
            cp0.wait()
            cp1.wait()
            pltpu.async_copy(rows[b][0], srcg_hbm.at[pl.ds(base, CH)],
                             semw[b])
            pltpu.async_copy(rows[b][1], dstg_hbm.at[pl.ds(base, CH)],
                             semw[b])

            @pl.when(j + 2 < NFULL)
            def _():
                nb = base_of(j + 2)
                pltpu.async_copy(ei0_hbm.at[pl.ds(nb, CH)], idx[b][0],
                                 semi[b])
                pltpu.async_copy(ei1_hbm.at[pl.ds(nb, CH)], idx[b][1],
                                 semi[b])

        return carry

    lax.fori_loop(0, NFULL // 2, outer, 0)

    for b in (0, 1):
        pv = base_of(NFULL - 2 + b)
        pltpu.make_async_copy(rows[b][0], srcg_hbm.at[pl.ds(pv, CH)],
                              semw[b]).wait()
        pltpu.make_async_copy(rows[b][1], dstg_hbm.at[pl.ds(pv, CH)],
                              semw[b]).wait()

    @pl.when(wid < NCH - NFULL * NW)
    def _():
        base = (NFULL * NW + wid) * CH
        pltpu.sync_copy(ei0_hbm.at[pl.ds(base, CH)], idx[0][0])
        pltpu.sync_copy(ei1_hbm.at[pl.ds(base, CH)], idx[0][1])
        cp0 = pltpu.async_copy(feats_hbm.at[idx[0][0]], rows[0][0], semg[0])
        cp1 = pltpu.async_copy(feats_hbm.at[idx[0][1]], rows[0][1], semg[0])
        cp0.wait()
        cp1.wait()
        pltpu.sync_copy(rows[0][0], srcg_hbm.at[pl.ds(base, CH)])
        pltpu.sync_copy(rows[0][1], dstg_hbm.at[pl.ds(base, CH)])


def _gather(feats, ei0, ei1):
    f = pl.kernel(
        _gather_body,
        out_type=(jax.ShapeDtypeStruct((E, D), jnp.float32),
                  jax.ShapeDtypeStruct((E, D), jnp.float32)),
        mesh=plsc.VectorSubcoreMesh(core_axis_name="c", subcore_axis_name="s"),
        scratch_types=[
            pltpu.VMEM((CH,), jnp.int32),
            pltpu.VMEM((CH,), jnp.int32),
            pltpu.VMEM((CH,), jnp.int32),
            pltpu.VMEM((CH,), jnp.int32),
            pltpu.VMEM((CH, D), jnp.float32),
            pltpu.VMEM((CH, D), jnp.float32),
            pltpu.VMEM((CH, D), jnp.float32),
            pltpu.VMEM((CH, D), jnp.float32),
            pltpu.SemaphoreType.DMA,
            pltpu.SemaphoreType.DMA,
            pltpu.SemaphoreType.DMA,
            pltpu.SemaphoreType.DMA,
            pltpu.SemaphoreType.DMA,
            pltpu.SemaphoreType.DMA,
        ],
    )
    return f(feats, ei0, ei1)


def _edge_body(src_ref, dst_ref, w1a_ref, w1b_ref, b1_ref, lng_ref, lnb_ref,
               w2_ref, b2_ref, h2_ref, s1_ref, s2_ref):
    src = src_ref[...]
    dst = dst_ref[...]
    multi = jnp.sum(src * dst, axis=-1, keepdims=True)
    dd = src - dst
    dist = jnp.sqrt(jnp.sum(dd * dd, axis=-1, keepdims=True))
    z = 2.0 * dist + 2.0 * (dist * dist * dist / 3.0 + multi * dist * dist)
    coef = 1.0 - jax.nn.sigmoid(z)
    h = (jnp.dot((1.0 + coef) * src, w1a_ref[...],
                 preferred_element_type=jnp.float32)
         + jnp.dot(dst, w1b_ref[...], preferred_element_type=jnp.float32)
         + b1_ref[...])
    h = _elu(h)
    mu = jnp.mean(h, axis=-1, keepdims=True)
    hc = h - mu
    var = jnp.mean(hc * hc, axis=-1, keepdims=True)
    h = hc / jnp.sqrt(var + 1e-5) * lng_ref[...] + lnb_ref[...]
    h = jnp.dot(h, w2_ref[...], preferred_element_type=jnp.float32) + b2_ref[...]
    h = _elu(h)
    h2_ref[...] = h

    @pl.when(pl.program_id(0) == 0)
    def _():
        s1_ref[...] = jnp.zeros_like(s1_ref)
        s2_ref[...] = jnp.zeros_like(s2_ref)

    s1_ref[...] += jnp.sum(h, axis=0, keepdims=True)
    s2_ref[...] += jnp.sum(h * h, axis=0, keepdims=True)


def _edge_mlp(srcg, dstg, w1aT, w1bT, b1, ln_g, ln_b, w2T, b2):
    full = pl.BlockSpec((D, D), lambda i: (0, 0))
    vec = pl.BlockSpec((1, D), lambda i: (0, 0))
    return pl.pallas_call(
        _edge_body,
        grid=(E // BE,),
        in_specs=[
            pl.BlockSpec((BE, D), lambda i: (i, 0)),
            pl.BlockSpec((BE, D), lambda i: (i, 0)),
            full, full, vec, vec, vec, full, vec,
        ],
        out_specs=[
            pl.BlockSpec((BE, D), lambda i: (i, 0)),
            vec, vec,
        ],
        out_shape=[
            jax.ShapeDtypeStruct((E, D), jnp.float32),
            jax.ShapeDtypeStruct((1, D), jnp.float32),
            jax.ShapeDtypeStruct((1, D), jnp.float32),
        ],
    )(srcg, dstg, w1aT, w1bT, b1, ln_g, ln_b, w2T, b2)


def _scatter_body(h2_hbm, ei1_hbm, zs_hbm, ones_hbm, sp_hbm, dg_hbm,
                  idxa, idxb, h2a, h2b, s_sh,
                  seml_a, seml_b, sems_a, sems_b):
    c = lax.axis_index("c")
    s = lax.axis_index("s")
    wid = c * NSUB + s
    r0 = s * ROWS_PER_SUB
    idx = (idxa, idxb)
    h2v = (h2a, h2b)
    seml = (seml_a, seml_b)
    sems = (sems_a, sems_b)
    nrem = NCH - NFULL * NW

    @pl.when(s == 0)
    def _():
        pltpu.sync_copy(zs_hbm, s_sh)

    plsc.subcore_barrier()

    def base_of(j):
        return (wid + NW * j) * CH

    pb0 = base_of(0)
    pltpu.async_copy(ei1_hbm.at[pl.ds(pb0, CH)], idx[0], seml[0])
    pltpu.async_copy(h2_hbm.at[pl.ds(pb0, CH)], h2v[0], seml[0])

    def outer(g, carry):
        for b in (0, 1):
            j = g * 2 + b
            base = base_of(j)
            pltpu.make_async_copy(
                ei1_hbm.at[pl.ds(base, CH)], idx[b], seml[b]).wait()
            pltpu.make_async_copy(
                h2_hbm.at[pl.ds(base, CH)], h2v[b], seml[b]).wait()
            pltpu.async_copy(h2v[b], s_sh.at[idx[b]], sems[b], add=True)

            o = 1 - b

            @pl.when(j >= 1)
            def _():
                pv = base_of(j - 1)
                pltpu.make_async_copy(
                    h2v[o], s_sh.at[idx[o]], sems[o]).wait()

            @pl.when(j + 1 < NFULL)
            def _():
                nb = base_of(j + 1)
                pltpu.async_copy(ei1_hbm.at[pl.ds(nb, CH)], idx[o], seml[o])
                pltpu.async_copy(h2_hbm.at[pl.ds(nb, CH)], h2v[o], seml[o])

        return carry

    lax.fori_loop(0, NFULL // 2, outer, 0)
    pltpu.make_async_copy(h2v[1], s_sh.at[idx[1]], sems[1]).wait()

    @pl.when(wid < nrem)
    def _():
        base = (NFULL * NW + wid) * CH
        pltpu.sync_copy(ei1_hbm.at[pl.ds(base, CH)], idx[0])
        pltpu.sync_copy(h2_hbm.at[pl.ds(base, CH)], h2v[0])
        pltpu.sync_copy(h2v[0], s_sh.at[idx[0]], add=True)

    plsc.subcore_barrier()
    pltpu.sync_copy(s_sh.at[pl.ds(r0, ROWS_PER_SUB)],
                    sp_hbm.at[pl.ds(c * NPAD + r0, ROWS_PER_SUB)])
    plsc.subcore_barrier()

    @pl.when(s == 0)
    def _():
        pltpu.sync_copy(zs_hbm, s_sh)

    pltpu.sync_copy(ones_hbm, h2a)
    plsc.subcore_barrier()

    ones_v = h2a
    pltpu.async_copy(ei1_hbm.at[pl.ds(pb0, CH)], idx[0], seml[0])

    def douter(g, carry):
        for b in (0, 1):
            j = g * 2 + b
            base = base_of(j)
            pltpu.make_async_copy(
                ei1_hbm.at[pl.ds(base, CH)], idx[b], seml[b]).wait()
            pltpu.async_copy(ones_v, s_sh.at[idx[b]], sems[b], add=True)

            o = 1 - b

            @pl.when(j >= 1)
            def _():
                pltpu.make_async_copy(
                    ones_v, s_sh.at[idx[o]], sems[o]).wait()

            @pl.when(j + 1 < NFULL)
            def _():
                nb = base_of(j + 1)
                pltpu.async_copy(ei1_hbm.at[pl.ds(nb, CH)], idx[o], seml[o])

        return carry

    lax.fori_loop(0, NFULL // 2, douter, 0)
    pltpu.make_async_copy(ones_v, s_sh.at[idx[1]], sems[1]).wait()

    @pl.when(wid < nrem)
    def _():
        base = (NFULL * NW + wid) * CH
        pltpu.sync_copy(ei1_hbm.at[pl.ds(base, CH)], idx[0])
        pltpu.sync_copy(ones_v, s_sh.at[idx[0]], add=True)

    plsc.subcore_barrier()
    pltpu.sync_copy(s_sh.at[pl.ds(r0, ROWS_PER_SUB)],
                    dg_hbm.at[pl.ds(c * NPAD + r0, ROWS_PER_SUB)])


def _scatter(h2, ei1, zs, ones):
    f = pl.kernel(
        _scatter_body,
        out_type=(jax.ShapeDtypeStruct((NCORE * NPAD, D), jnp.float32),
                  jax.ShapeDtypeStruct((NCORE * NPAD, D), jnp.float32)),
        mesh=plsc.VectorSubcoreMesh(core_axis_name="c", subcore_axis_name="s"),
        scratch_types=[
            pltpu.VMEM((CH,), jnp.int32),
            pltpu.VMEM((CH,), jnp.int32),
            pltpu.VMEM((CH, D), jnp.float32),
            pltpu.VMEM((CH, D), jnp.float32),
            pltpu.VMEM_SHARED((NPAD, D), jnp.float32),
            pltpu.SemaphoreType.DMA,
            pltpu.SemaphoreType.DMA,
            pltpu.SemaphoreType.DMA,
            pltpu.SemaphoreType.DMA,
        ],
    )
    return f(h2, ei1, zs, ones)


def _final_body(sp_ref, dg_ref, s1_ref, s2_ref, bng_ref, bnb_ref, wo_ref,
                bo_ref, feats_ref, o_ref):
    s_sum = sp_ref[0] + sp_ref[1]
    d_sum = dg_ref[0] + dg_ref[1]
    m = s1_ref[...] / float(E)
    v = s2_ref[...] / float(E) - m * m
    a = bng_ref[...] / jnp.sqrt(v + 1e-5)
    cv = bnb_ref[...] - m * a
    kv = jnp.dot(cv, wo_ref[...], preferred_element_type=jnp.float32)
    out = (jnp.dot(s_sum * a, wo_ref[...], preferred_element_type=jnp.float32)
           + d_sum * kv + bo_ref[...])
    out = _hyp(out)
    out = 1.0507009873554805 * jnp.where(
        out > 0, out, 1.6732632423543772 * (jnp.exp(jnp.minimum(out, 0.0)) - 1.0))
    o_ref[...] = out + feats_ref[...]


def _final(sp, dg, s1, s2, bn_g, bn_b, woT, bo, feats):
    vec = pl.BlockSpec((1, D), lambda i: (0, 0))
    return pl.pallas_call(
        _final_body,
        grid=(N // BN_,),
        in_specs=[
            pl.BlockSpec((NCORE, BN_, D), lambda i: (0, i, 0)),
            pl.BlockSpec((NCORE, BN_, D), lambda i: (0, i, 0)),
            vec, vec, vec, vec,
            pl.BlockSpec((D, D), lambda i: (0, 0)),
            vec,
            pl.BlockSpec((BN_, D), lambda i: (i, 0)),
        ],
        out_specs=pl.BlockSpec((BN_, D), lambda i: (i, 0)),
        out_shape=jax.ShapeDtypeStruct((N, D), jnp.float32),
    )(sp, dg, s1, s2, bn_g, bn_b, woT, bo, feats)


def kernel(features, edge_index, c, W1, b1, ln_g, ln_b, W2, b2, bn_g, bn_b,
           Wo, bo):
    del c
    f32 = jnp.float32

    feats = _node_transform(features)

    ei0 = edge_index[0]
    ei1 = edge_index[1]
    srcg, dstg = _gather(feats, ei0, ei1)

    h2, s1, s2 = _edge_mlp(
        srcg, dstg,
        W1[:, :D].T, W1[:, D:].T, b1[None], ln_g[None], ln_b[None],
        W2.T, b2[None])

    zs = jnp.zeros((NPAD, D), f32)
    ones = jnp.ones((CH, D), f32)
    sp, dg = _scatter(h2, ei1, zs, ones)
    sp = sp.reshape(NCORE, NPAD, D)[:, :N]
    dg = dg.reshape(NCORE, NPAD, D)[:, :N]
    return _final(sp, dg, s1, s2, bn_g[None], bn_b[None], Wo.T,
                  bo[None], feats)


# speedup vs baseline: 4.8454x; 1.2801x over previous
import jax
import jax.numpy as jnp
from jax import lax
from jax.experimental import pallas as pl
from jax.experimental.pallas import tpu as pltpu
from jax.experimental.pallas import tpu_sc as plsc

N = 10000
E = 320000
D = 128

CH = 128
NCH = E // CH
NSUB = 16
NCORE = 2
NW = NCORE * NSUB
NPAD = 10240
ROWS_PER_SUB = NPAD // NSUB

BE = 2560
BN_ = 2000


def _norm(x):
    return jnp.maximum(jnp.sqrt(jnp.sum(x * x, axis=-1, keepdims=True)), 1e-15)


def _hyp(x):
    n = _norm(x)
    e = jnp.tanh(n) * x / n
    ne = _norm(e)
    maxn = 1.0 - 1e-05
    e = jnp.where(ne > maxn, e / ne * maxn, e)
    n3 = _norm(e)
    atanh = 0.5 * (jnp.log1p(n3) - jnp.log1p(-n3))
    return atanh / n3 * e


def _elu(x):
    return jnp.where(x > 0, x, jnp.exp(jnp.minimum(x, 0.0)) - 1.0)


def _node_body(x_ref, o_ref):
    o_ref[...] = _hyp(x_ref[...])


def _node_transform(features):
    blk = pl.BlockSpec((BN_, D), lambda i: (i, 0))
    return pl.pallas_call(
        _node_body,
        grid=(N // BN_,),
        in_specs=[blk],
        out_specs=blk,
        out_shape=jax.ShapeDtypeStruct((N, D), jnp.float32),
    )(features)


NFULL = NCH // NW

SLICE_A = (0, 40, 0)
SLICE_B = (1280, 38, 4)


def _make_gather_body(c0, nfull, nrem):
    def body(feats_hbm, ei0_hbm, ei1_hbm, srcg_hbm, dstg_hbm,
             idx0a, idx1a, idx0b, idx1b, r0a, r1a, r0b, r1b,
             semi_a, semi_b, semg_a, semg_b, semw_a, semw_b):
        c = lax.axis_index("c")
        s = lax.axis_index("s")
        wid = c * NSUB + s
        idx = ((idx0a, idx1a), (idx0b, idx1b))
        rows = ((r0a, r1a), (r0b, r1b))
        semi = (semi_a, semi_b)
        semg = (semg_a, semg_b)
        semw = (semw_a, semw_b)

        def gbase(j):
            return (c0 + wid + NW * j) * CH

        def lbase(j):
            return (wid + NW * j) * CH

        for b in (0, 1):
            pb = gbase(b)
            pltpu.async_copy(ei0_hbm.at[pl.ds(pb, CH)], idx[b][0], semi[b])
            pltpu.async_copy(ei1_hbm.at[pl.ds(pb, CH)], idx[b][1], semi[b])

        def outer(g, carry):
            for b in (0, 1):
                j = g * 2 + b
                base = gbase(j)
                lb = lbase(j)

                @pl.when(j >= 2)
                def _():
                    pv = lbase(j - 2)
                    pltpu.make_async_copy(
                        rows[b][0], srcg_hbm.at[pl.ds(pv, CH)],
                        semw[b]).wait()
                    pltpu.make_async_copy(
                        rows[b][1], dstg_hbm.at[pl.ds(pv, CH)],
                        semw[b]).wait()

                pltpu.make_async_copy(
                    ei0_hbm.at[pl.ds(base, CH)], idx[b][0], semi[b]).wait()
                pltpu.make_async_copy(
                    ei1_hbm.at[pl.ds(base, CH)], idx[b][1], semi[b]).wait()
                cp0 = pltpu.async_copy(feats_hbm.at[idx[b][0]], rows[b][0],
                                       semg[b])
                cp1 = pltpu.async_copy(feats_hbm.at[idx[b][1]], rows[b][1],
                                       semg[b])
                cp0.wait()
                cp1.wait()
                pltpu.async_copy(rows[b][0], srcg_hbm.at[pl.ds(lb, CH)],
                                 semw[b])
                pltpu.async_copy(rows[b][1], dstg_hbm.at[pl.ds(lb, CH)],
                                 semw[b])

                @pl.when(j + 2 < nfull)
                def _():
                    nb = gbase(j + 2)
                    pltpu.async_copy(ei0_hbm.at[pl.ds(nb, CH)], idx[b][0],
                                     semi[b])
                    pltpu.async_copy(ei1_hbm.at[pl.ds(nb, CH)], idx[b][1],
                                     semi[b])

            return carry

        lax.fori_loop(0, nfull // 2, outer, 0)

        for b in (0, 1):
            pv = lbase(nfull - 2 + b)
            pltpu.make_async_copy(rows[b][0], srcg_hbm.at[pl.ds(pv, CH)],
                                  semw[b]).wait()
            pltpu.make_async_copy(rows[b][1], dstg_hbm.at[pl.ds(pv, CH)],
                                  semw[b]).wait()

        if nrem:
            @pl.when(wid < nrem)
            def _():
                base = (c0 + nfull * NW + wid) * CH
                lb = (nfull * NW + wid) * CH
                pltpu.sync_copy(ei0_hbm.at[pl.ds(base, CH)], idx[0][0])
                pltpu.sync_copy(ei1_hbm.at[pl.ds(base, CH)], idx[0][1])
                cp0 = pltpu.async_copy(feats_hbm.at[idx[0][0]], rows[0][0],
                                       semg[0])
                cp1 = pltpu.async_copy(feats_hbm.at[idx[0][1]], rows[0][1],
                                       semg[0])
                cp0.wait()
                cp1.wait()
                pltpu.sync_copy(rows[0][0], srcg_hbm.at[pl.ds(lb, CH)])
                pltpu.sync_copy(rows[0][1], dstg_hbm.at[pl.ds(lb, CH)])

    return body


def _make_gather(sl):
    c0, nfull, nrem = sl
    es = (nfull * NW + nrem) * CH
    return pl.kernel(
        _make_gather_body(c0, nfull, nrem),
        out_type=(jax.ShapeDtypeStruct((es, D), jnp.float32),
                  jax.ShapeDtypeStruct((es, D), jnp.float32)),
        mesh=plsc.VectorSubcoreMesh(core_axis_name="c", subcore_axis_name="s"),
        scratch_types=[
            pltpu.VMEM((CH,), jnp.int32),
            pltpu.VMEM((CH,), jnp.int32),
            pltpu.VMEM((CH,), jnp.int32),
            pltpu.VMEM((CH,), jnp.int32),
            pltpu.VMEM((CH, D), jnp.float32),
            pltpu.VMEM((CH, D), jnp.float32),
            pltpu.VMEM((CH, D), jnp.float32),
            pltpu.VMEM((CH, D), jnp.float32),
            pltpu.SemaphoreType.DMA,
            pltpu.SemaphoreType.DMA,
            pltpu.SemaphoreType.DMA,
            pltpu.SemaphoreType.DMA,
            pltpu.SemaphoreType.DMA,
            pltpu.SemaphoreType.DMA,
        ],
    )


def _edge_body(src_ref, dst_ref, w1a_ref, w1b_ref, b1_ref, lng_ref, lnb_ref,
               w2_ref, b2_ref, h2_ref, s1_ref, s2_ref):
    src = src_ref[...].astype(jnp.float32)
    dst = dst_ref[...].astype(jnp.float32)
    multi = jnp.sum(src * dst, axis=-1, keepdims=True)
    dd = src - dst
    dist = jnp.sqrt(jnp.sum(dd * dd, axis=-1, keepdims=True))
    z = 2.0 * dist + 2.0 * (dist * dist * dist / 3.0 + multi * dist * dist)
    coef = 1.0 - jax.nn.sigmoid(z)
    h = (jnp.dot((1.0 + coef) * src, w1a_ref[...],
                 preferred_element_type=jnp.float32)
         + jnp.dot(dst, w1b_ref[...], preferred_element_type=jnp.float32)
         + b1_ref[...])
    h = _elu(h)
    mu = jnp.mean(h, axis=-1, keepdims=True)
    hc = h - mu
    var = jnp.mean(hc * hc, axis=-1, keepdims=True)
    h = hc / jnp.sqrt(var + 1e-5) * lng_ref[...] + lnb_ref[...]
    h = jnp.dot(h, w2_ref[...], preferred_element_type=jnp.float32) + b2_ref[...]
    h = _elu(h)
    h2_ref[...] = h

    @pl.when(pl.program_id(0) == 0)
    def _():
        s1_ref[...] = jnp.zeros_like(s1_ref)
        s2_ref[...] = jnp.zeros_like(s2_ref)

    s1_ref[...] += jnp.sum(h, axis=0, keepdims=True)
    s2_ref[...] += jnp.sum(h * h, axis=0, keepdims=True)


def _edge_mlp(srcg, dstg, w1aT, w1bT, b1, ln_g, ln_b, w2T, b2):
    full = pl.BlockSpec((D, D), lambda i: (0, 0))
    vec = pl.BlockSpec((1, D), lambda i: (0, 0))
    return pl.pallas_call(
        _edge_body,
        grid=(srcg.shape[0] // BE,),
        in_specs=[
            pl.BlockSpec((BE, D), lambda i: (i, 0)),
            pl.BlockSpec((BE, D), lambda i: (i, 0)),
            full, full, vec, vec, vec, full, vec,
        ],
        out_specs=[
            pl.BlockSpec((BE, D), lambda i: (i, 0)),
            vec, vec,
        ],
        out_shape=[
            jax.ShapeDtypeStruct((srcg.shape[0], D), jnp.float32),
            jax.ShapeDtypeStruct((1, D), jnp.float32),
            jax.ShapeDtypeStruct((1, D), jnp.float32),
        ],
    )(srcg, dstg, w1aT, w1bT, b1, ln_g, ln_b, w2T, b2)


def _make_scatter_body(c0, nfull, nrem, with_deg):
    def body(h2_hbm, ei1_hbm, zs_hbm, ones_hbm, *refs):
        if with_deg:
            (sp_hbm, dg_hbm, idxa, idxb, h2a, h2b, s_sh,
             seml_a, seml_b, sems_a, sems_b) = refs
        else:
            (sp_hbm, idxa, idxb, h2a, h2b, s_sh,
             seml_a, seml_b, sems_a, sems_b) = refs
        c = lax.axis_index("c")
        s = lax.axis_index("s")
        wid = c * NSUB + s
        r0 = s * ROWS_PER_SUB
        idx = (idxa, idxb)
        h2v = (h2a, h2b)
        seml = (seml_a, seml_b)
        sems = (sems_a, sems_b)

        @pl.when(s == 0)
        def _():
            pltpu.sync_copy(zs_hbm, s_sh)

        plsc.subcore_barrier()

        def gbase(j):
            return (c0 + wid + NW * j) * CH

        def lbase(j):
            return (wid + NW * j) * CH

        pltpu.async_copy(ei1_hbm.at[pl.ds(gbase(0), CH)], idx[0], seml[0])
        pltpu.async_copy(h2_hbm.at[pl.ds(lbase(0), CH)], h2v[0], seml[0])

        def outer(g, carry):
            for b in (0, 1):
                j = g * 2 + b
                pltpu.make_async_copy(
                    ei1_hbm.at[pl.ds(gbase(j), CH)], idx[b], seml[b]).wait()
                pltpu.make_async_copy(
                    h2_hbm.at[pl.ds(lbase(j), CH)], h2v[b], seml[b]).wait()
                pltpu.async_copy(h2v[b], s_sh.at[idx[b]], sems[b], add=True)

                o = 1 - b

                @pl.when(j >= 1)
                def _():
                    pltpu.make_async_copy(
                        h2v[o], s_sh.at[idx[o]], sems[o]).wait()

                @pl.when(j + 1 < nfull)
                def _():
                    pltpu.async_copy(ei1_hbm.at[pl.ds(gbase(j + 1), CH)],
                                     idx[o], seml[o])
                    pltpu.async_copy(h2_hbm.at[pl.ds(lbase(j + 1), CH)],
                                     h2v[o], seml[o])

            return carry

        lax.fori_loop(0, nfull // 2, outer, 0)
        pltpu.make_async_copy(h2v[1], s_sh.at[idx[1]], sems[1]).wait()

        if nrem:
            @pl.when(wid < nrem)
            def _():
                gb = (c0 + nfull * NW + wid) * CH
                lb = (nfull * NW + wid) * CH
                pltpu.sync_copy(ei1_hbm.at[pl.ds(gb, CH)], idx[0])
                pltpu.sync_copy(h2_hbm.at[pl.ds(lb, CH)], h2v[0])
                pltpu.sync_copy(h2v[0], s_sh.at[idx[0]], add=True)

        plsc.subcore_barrier()
        pltpu.sync_copy(s_sh.at[pl.ds(r0, ROWS_PER_SUB)],
                        sp_hbm.at[pl.ds(c * NPAD + r0, ROWS_PER_SUB)])

        if not with_deg:
            return

        plsc.subcore_barrier()

        @pl.when(s == 0)
        def _():
            pltpu.sync_copy(zs_hbm, s_sh)

        pltpu.sync_copy(ones_hbm, h2a)
        plsc.subcore_barrier()

        ones_v = h2a

        def dgbase(j):
            return (wid + NW * j) * CH

        pltpu.async_copy(ei1_hbm.at[pl.ds(dgbase(0), CH)], idx[0], seml[0])

        def douter(g, carry):
            for b in (0, 1):
                j = g * 2 + b
                pltpu.make_async_copy(
                    ei1_hbm.at[pl.ds(dgbase(j), CH)], idx[b], seml[b]).wait()
                pltpu.async_copy(ones_v, s_sh.at[idx[b]], sems[b], add=True)

                o = 1 - b

                @pl.when(j >= 1)
                def _():
                    pltpu.make_async_copy(
                        ones_v, s_sh.at[idx[o]], sems[o]).wait()

                @pl.when(j + 1 < NFULL)
                def _():
                    pltpu.async_copy(ei1_hbm.at[pl.ds(dgbase(j + 1), CH)],
                                     idx[o], seml[o])

            return carry

        lax.fori_loop(0, NFULL // 2, douter, 0)
        pltpu.make_async_copy(ones_v, s_sh.at[idx[1]], sems[1]).wait()

        @pl.when(wid < NCH - NFULL * NW)
        def _():
            gb = (NFULL * NW + wid) * CH
            pltpu.sync_copy(ei1_hbm.at[pl.ds(gb, CH)], idx[0])
            pltpu.sync_copy(ones_v, s_sh.at[idx[0]], add=True)

        plsc.subcore_barrier()
        pltpu.sync_copy(s_sh.at[pl.ds(r0, ROWS_PER_SUB)],
                        dg_hbm.at[pl.ds(c * NPAD + r0, ROWS_PER_SUB)])

    return body


def _make_scatter(sl, with_deg):
    c0, nfull, nrem = sl
    acc = jax.ShapeDtypeStruct((NCORE * NPAD, D), jnp.float32)
    return pl.kernel(
        _make_scatter_body(c0, nfull, nrem, with_deg),
        out_type=(acc, acc) if with_deg else acc,
        mesh=plsc.VectorSubcoreMesh(core_axis_name="c", subcore_axis_name="s"),
        scratch_types=[
            pltpu.VMEM((CH,), jnp.int32),
            pltpu.VMEM((CH,), jnp.int32),
            pltpu.VMEM((CH, D), jnp.float32),
            pltpu.VMEM((CH, D), jnp.float32),
            pltpu.VMEM_SHARED((NPAD, D), jnp.float32),
            pltpu.SemaphoreType.DMA,
            pltpu.SemaphoreType.DMA,
            pltpu.SemaphoreType.DMA,
            pltpu.SemaphoreType.DMA,
        ],
    )


def _final_body(spa_ref, spb_ref, dg_ref, s1a_ref, s2a_ref, s1b_ref,
                s2b_ref, bng_ref, bnb_ref, wo_ref, bo_ref, feats_ref, o_ref):
    s_sum = spa_ref[0] + spa_ref[1] + spb_ref[0] + spb_ref[1]
    d_sum = dg_ref[0] + dg_ref[1]
    m = (s1a_ref[...] + s1b_ref[...]) / float(E)
    v = (s2a_ref[...] + s2b_ref[...]) / float(E) - m * m
    a = bng_ref[...] / jnp.sqrt(v + 1e-5)
    cv = bnb_ref[...] - m * a
    kv = jnp.dot(cv, wo_ref[...], preferred_element_type=jnp.float32)
    out = (jnp.dot(s_sum * a, wo_ref[...], preferred_element_type=jnp.float32)
           + d_sum * kv + bo_ref[...])
    out = _hyp(out)
    out = 1.0507009873554805 * jnp.where(
        out > 0, out, 1.6732632423543772 * (jnp.exp(jnp.minimum(out, 0.0)) - 1.0))
    o_ref[...] = out + feats_ref[...]


def _final(spa, spb, dg, s1a, s2a, s1b, s2b, bn_g, bn_b, woT, bo, feats):
    vec = pl.BlockSpec((1, D), lambda i: (0, 0))
    acc = pl.BlockSpec((NCORE, BN_, D), lambda i: (0, i, 0))
    return pl.pallas_call(
        _final_body,
        grid=(N // BN_,),
        in_specs=[
            acc, acc, acc,
            vec, vec, vec, vec, vec, vec,
            pl.BlockSpec((D, D), lambda i: (0, 0)),
            vec,
            pl.BlockSpec((BN_, D), lambda i: (i, 0)),
        ],
        out_specs=pl.BlockSpec((BN_, D), lambda i: (i, 0)),
        out_shape=jax.ShapeDtypeStruct((N, D), jnp.float32),
    )(spa, spb, dg, s1a, s2a, s1b, s2b, bn_g, bn_b, woT, bo, feats)


def kernel(features, edge_index, c, W1, b1, ln_g, ln_b, W2, b2, bn_g, bn_b,
           Wo, bo):
    del c
    f32 = jnp.float32

    feats = _node_transform(features)

    ei0 = edge_index[0]
    ei1 = edge_index[1]
    gA = _make_gather(SLICE_A)
    gB = _make_gather(SLICE_B)
    srcgA, dstgA = gA(feats, ei0, ei1)
    srcgB, dstgB = gB(feats, ei0, ei1)

    w1aT = W1[:, :D].T
    w1bT = W1[:, D:].T
    w2T = W2.T
    h2A, s1A, s2A = _edge_mlp(srcgA, dstgA, w1aT, w1bT, b1[None], ln_g[None],
                              ln_b[None], w2T, b2[None])
    h2B, s1B, s2B = _edge_mlp(srcgB, dstgB, w1aT, w1bT, b1[None], ln_g[None],
                              ln_b[None], w2T, b2[None])

    zs = jnp.zeros((NPAD, D), f32)
    ones = jnp.ones((CH, D), f32)
    spA, dg = _make_scatter(SLICE_A, True)(h2A, ei1, zs, ones)
    spB = _make_scatter(SLICE_B, False)(h2B, ei1, zs, ones)
    spA = spA.reshape(NCORE, NPAD, D)[:, :N]
    spB = spB.reshape(NCORE, NPAD, D)[:, :N]
    dg = dg.reshape(NCORE, NPAD, D)[:, :N]
    return _final(spA, spB, dg, s1A, s2A, s1B, s2B, bn_g[None], bn_b[None],
                  Wo.T, bo[None], feats)


# speedup vs baseline: 5.1138x; 1.0554x over previous
import jax
import jax.numpy as jnp
from jax import lax
from jax.experimental import pallas as pl
from jax.experimental.pallas import tpu as pltpu
from jax.experimental.pallas import tpu_sc as plsc

N = 10000
E = 320000
D = 128

CH = 128
NCH = E // CH
NSUB = 16
NCORE = 2
NW = NCORE * NSUB
NPAD = 10240
ROWS_PER_SUB = NPAD // NSUB

BE = 2560
BN_ = 2000


def _norm(x):
    return jnp.maximum(jnp.sqrt(jnp.sum(x * x, axis=-1, keepdims=True)), 1e-15)


def _hyp(x):
    n = _norm(x)
    e = jnp.tanh(n) * x / n
    ne = _norm(e)
    maxn = 1.0 - 1e-05
    e = jnp.where(ne > maxn, e / ne * maxn, e)
    n3 = _norm(e)
    atanh = 0.5 * (jnp.log1p(n3) - jnp.log1p(-n3))
    return atanh / n3 * e


def _elu(x):
    return jnp.where(x > 0, x, jnp.exp(jnp.minimum(x, 0.0)) - 1.0)


def _node_body(x_ref, o_ref):
    o_ref[...] = _hyp(x_ref[...])


def _node_transform(features):
    blk = pl.BlockSpec((BN_, D), lambda i: (i, 0))
    return pl.pallas_call(
        _node_body,
        grid=(N // BN_,),
        in_specs=[blk],
        out_specs=blk,
        out_shape=jax.ShapeDtypeStruct((N, D), jnp.float32),
    )(features)


NFULL = NCH // NW

SLICES = ((0, 20, 0),
          (640, 40, 0),
          (1920, 18, 4))


def _make_gather_body(c0, nfull, nrem):
    def body(feats_hbm, ei0_hbm, ei1_hbm, srcg_hbm, dstg_hbm,
             idx0a, idx1a, idx0b, idx1b, r0a, r1a, r0b, r1b,
             semi_a, semi_b, semg_a, semg_b, semw_a, semw_b):
        c = lax.axis_index("c")
        s = lax.axis_index("s")
        wid = c * NSUB + s
        idx = ((idx0a, idx1a), (idx0b, idx1b))
        rows = ((r0a, r1a), (r0b, r1b))
        semi = (semi_a, semi_b)
        semg = (semg_a, semg_b)
        semw = (semw_a, semw_b)

        def gbase(j):
            return (c0 + wid + NW * j) * CH

        def lbase(j):
            return (wid + NW * j) * CH

        for b in (0, 1):
            pb = gbase(b)
            pltpu.async_copy(ei0_hbm.at[pl.ds(pb, CH)], idx[b][0], semi[b])
            pltpu.async_copy(ei1_hbm.at[pl.ds(pb, CH)], idx[b][1], semi[b])

        def outer(g, carry):
            for b in (0, 1):
                j = g * 2 + b
                base = gbase(j)
                lb = lbase(j)

                @pl.when(j >= 2)
                def _():
                    pv = lbase(j - 2)
                    pltpu.make_async_copy(
                        rows[b][0], srcg_hbm.at[pl.ds(pv, CH)],
                        semw[b]).wait()
                    pltpu.make_async_copy(
                        rows[b][1], dstg_hbm.at[pl.ds(pv, CH)],
                        semw[b]).wait()

                pltpu.make_async_copy(
                    ei0_hbm.at[pl.ds(base, CH)], idx[b][0], semi[b]).wait()
                pltpu.make_async_copy(
                    ei1_hbm.at[pl.ds(base, CH)], idx[b][1], semi[b]).wait()
                cp0 = pltpu.async_copy(feats_hbm.at[idx[b][0]], rows[b][0],
                                       semg[b])
                cp1 = pltpu.async_copy(feats_hbm.at[idx[b][1]], rows[b][1],
                                       semg[b])
                cp0.wait()
                cp1.wait()
                pltpu.async_copy(rows[b][0], srcg_hbm.at[pl.ds(lb, CH)],
                                 semw[b])
                pltpu.async_copy(rows[b][1], dstg_hbm.at[pl.ds(lb, CH)],
                                 semw[b])

                @pl.when(j + 2 < nfull)
                def _():
                    nb = gbase(j + 2)
                    pltpu.async_copy(ei0_hbm.at[pl.ds(nb, CH)], idx[b][0],
                                     semi[b])
                    pltpu.async_copy(ei1_hbm.at[pl.ds(nb, CH)], idx[b][1],
                                     semi[b])

            return carry

        lax.fori_loop(0, nfull // 2, outer, 0)

        for b in (0, 1):
            pv = lbase(nfull - 2 + b)
            pltpu.make_async_copy(rows[b][0], srcg_hbm.at[pl.ds(pv, CH)],
                                  semw[b]).wait()
            pltpu.make_async_copy(rows[b][1], dstg_hbm.at[pl.ds(pv, CH)],
                                  semw[b]).wait()

        if nrem:
            @pl.when(wid < nrem)
            def _():
                base = (c0 + nfull * NW + wid) * CH
                lb = (nfull * NW + wid) * CH
                pltpu.sync_copy(ei0_hbm.at[pl.ds(base, CH)], idx[0][0])
                pltpu.sync_copy(ei1_hbm.at[pl.ds(base, CH)], idx[0][1])
                cp0 = pltpu.async_copy(feats_hbm.at[idx[0][0]], rows[0][0],
                                       semg[0])
                cp1 = pltpu.async_copy(feats_hbm.at[idx[0][1]], rows[0][1],
                                       semg[0])
                cp0.wait()
                cp1.wait()
                pltpu.sync_copy(rows[0][0], srcg_hbm.at[pl.ds(lb, CH)])
                pltpu.sync_copy(rows[0][1], dstg_hbm.at[pl.ds(lb, CH)])

    return body


def _make_gather(sl):
    c0, nfull, nrem = sl
    es = (nfull * NW + nrem) * CH
    return pl.kernel(
        _make_gather_body(c0, nfull, nrem),
        out_type=(jax.ShapeDtypeStruct((es, D), jnp.float32),
                  jax.ShapeDtypeStruct((es, D), jnp.float32)),
        mesh=plsc.VectorSubcoreMesh(core_axis_name="c", subcore_axis_name="s"),
        scratch_types=[
            pltpu.VMEM((CH,), jnp.int32),
            pltpu.VMEM((CH,), jnp.int32),
            pltpu.VMEM((CH,), jnp.int32),
            pltpu.VMEM((CH,), jnp.int32),
            pltpu.VMEM((CH, D), jnp.float32),
            pltpu.VMEM((CH, D), jnp.float32),
            pltpu.VMEM((CH, D), jnp.float32),
            pltpu.VMEM((CH, D), jnp.float32),
            pltpu.SemaphoreType.DMA,
            pltpu.SemaphoreType.DMA,
            pltpu.SemaphoreType.DMA,
            pltpu.SemaphoreType.DMA,
            pltpu.SemaphoreType.DMA,
            pltpu.SemaphoreType.DMA,
        ],
    )


def _edge_body(src_ref, dst_ref, w1a_ref, w1b_ref, b1_ref, lng_ref, lnb_ref,
               w2_ref, b2_ref, h2_ref, s1_ref, s2_ref):
    src = src_ref[...].astype(jnp.float32)
    dst = dst_ref[...].astype(jnp.float32)
    multi = jnp.sum(src * dst, axis=-1, keepdims=True)
    dd = src - dst
    dist = jnp.sqrt(jnp.sum(dd * dd, axis=-1, keepdims=True))
    z = 2.0 * dist + 2.0 * (dist * dist * dist / 3.0 + multi * dist * dist)
    coef = 1.0 - jax.nn.sigmoid(z)
    h = (jnp.dot((1.0 + coef) * src, w1a_ref[...],
                 preferred_element_type=jnp.float32)
         + jnp.dot(dst, w1b_ref[...], preferred_element_type=jnp.float32)
         + b1_ref[...])
    h = _elu(h)
    mu = jnp.mean(h, axis=-1, keepdims=True)
    hc = h - mu
    var = jnp.mean(hc * hc, axis=-1, keepdims=True)
    h = hc / jnp.sqrt(var + 1e-5) * lng_ref[...] + lnb_ref[...]
    h = jnp.dot(h, w2_ref[...], preferred_element_type=jnp.float32) + b2_ref[...]
    h = _elu(h)
    h2_ref[...] = h

    @pl.when(pl.program_id(0) == 0)
    def _():
        s1_ref[...] = jnp.zeros_like(s1_ref)
        s2_ref[...] = jnp.zeros_like(s2_ref)

    s1_ref[...] += jnp.sum(h, axis=0, keepdims=True)
    s2_ref[...] += jnp.sum(h * h, axis=0, keepdims=True)


def _edge_mlp(srcg, dstg, w1aT, w1bT, b1, ln_g, ln_b, w2T, b2):
    full = pl.BlockSpec((D, D), lambda i: (0, 0))
    vec = pl.BlockSpec((1, D), lambda i: (0, 0))
    return pl.pallas_call(
        _edge_body,
        grid=(srcg.shape[0] // BE,),
        in_specs=[
            pl.BlockSpec((BE, D), lambda i: (i, 0)),
            pl.BlockSpec((BE, D), lambda i: (i, 0)),
            full, full, vec, vec, vec, full, vec,
        ],
        out_specs=[
            pl.BlockSpec((BE, D), lambda i: (i, 0)),
            vec, vec,
        ],
        out_shape=[
            jax.ShapeDtypeStruct((srcg.shape[0], D), jnp.float32),
            jax.ShapeDtypeStruct((1, D), jnp.float32),
            jax.ShapeDtypeStruct((1, D), jnp.float32),
        ],
    )(srcg, dstg, w1aT, w1bT, b1, ln_g, ln_b, w2T, b2)


def _make_scatter_body(c0, nfull, nrem):
    def body(h2_hbm, ei1_hbm, zs_hbm, sp_hbm,
             idxa, idxb, h2a, h2b, s_sh,
             seml_a, seml_b, sems_a, sems_b):
        c = lax.axis_index("c")
        s = lax.axis_index("s")
        wid = c * NSUB + s
        r0 = s * ROWS_PER_SUB
        idx = (idxa, idxb)
        h2v = (h2a, h2b)
        seml = (seml_a, seml_b)
        sems = (sems_a, sems_b)

        @pl.when(s == 0)
        def _():
            pltpu.sync_copy(zs_hbm, s_sh)

        plsc.subcore_barrier()

        def gbase(j):
            return (c0 + wid + NW * j) * CH

        def lbase(j):
            return (wid + NW * j) * CH

        pltpu.async_copy(ei1_hbm.at[pl.ds(gbase(0), CH)], idx[0], seml[0])
        pltpu.async_copy(h2_hbm.at[pl.ds(lbase(0), CH)], h2v[0], seml[0])

        def outer(g, carry):
            for b in (0, 1):
                j = g * 2 + b
                pltpu.make_async_copy(
                    ei1_hbm.at[pl.ds(gbase(j), CH)], idx[b], seml[b]).wait()
                pltpu.make_async_copy(
                    h2_hbm.at[pl.ds(lbase(j), CH)], h2v[b], seml[b]).wait()
                pltpu.async_copy(h2v[b], s_sh.at[idx[b]], sems[b], add=True)

                o = 1 - b

                @pl.when(j >= 1)
                def _():
                    pltpu.make_async_copy(
                        h2v[o], s_sh.at[idx[o]], sems[o]).wait()

                @pl.when(j + 1 < nfull)
                def _():
                    pltpu.async_copy(ei1_hbm.at[pl.ds(gbase(j + 1), CH)],
                                     idx[o], seml[o])
                    pltpu.async_copy(h2_hbm.at[pl.ds(lbase(j + 1), CH)],
                                     h2v[o], seml[o])

            return carry

        lax.fori_loop(0, nfull // 2, outer, 0)
        pltpu.make_async_copy(h2v[1], s_sh.at[idx[1]], sems[1]).wait()

        if nrem:
            @pl.when(wid < nrem)
            def _():
                gb = (c0 + nfull * NW + wid) * CH
                lb = (nfull * NW + wid) * CH
                pltpu.sync_copy(ei1_hbm.at[pl.ds(gb, CH)], idx[0])
                pltpu.sync_copy(h2_hbm.at[pl.ds(lb, CH)], h2v[0])
                pltpu.sync_copy(h2v[0], s_sh.at[idx[0]], add=True)

        plsc.subcore_barrier()
        pltpu.sync_copy(s_sh.at[pl.ds(r0, ROWS_PER_SUB)],
                        sp_hbm.at[pl.ds(c * NPAD + r0, ROWS_PER_SUB)])

    return body


def _make_scatter(sl):
    c0, nfull, nrem = sl
    return pl.kernel(
        _make_scatter_body(c0, nfull, nrem),
        out_type=jax.ShapeDtypeStruct((NCORE * NPAD, D), jnp.float32),
        mesh=plsc.VectorSubcoreMesh(core_axis_name="c", subcore_axis_name="s"),
        scratch_types=[
            pltpu.VMEM((CH,), jnp.int32),
            pltpu.VMEM((CH,), jnp.int32),
            pltpu.VMEM((CH, D), jnp.float32),
            pltpu.VMEM((CH, D), jnp.float32),
            pltpu.VMEM_SHARED((NPAD, D), jnp.float32),
            pltpu.SemaphoreType.DMA,
            pltpu.SemaphoreType.DMA,
            pltpu.SemaphoreType.DMA,
            pltpu.SemaphoreType.DMA,
        ],
    )


def _deg_body(ei1_hbm, zs_hbm, ones_hbm, dg_hbm,
              idxa, idxb, ones_v, s_sh, seml_a, seml_b, sems_a, sems_b):
    c = lax.axis_index("c")
    s = lax.axis_index("s")
    wid = c * NSUB + s
    r0 = s * ROWS_PER_SUB
    idx = (idxa, idxb)
    seml = (seml_a, seml_b)
    sems = (sems_a, sems_b)

    @pl.when(s == 0)
    def _():
        pltpu.sync_copy(zs_hbm, s_sh)

    pltpu.sync_copy(ones_hbm, ones_v)
    plsc.subcore_barrier()

    def gbase(j):
        return (wid + NW * j) * CH

    pltpu.async_copy(ei1_hbm.at[pl.ds(gbase(0), CH)], idx[0], seml[0])

    def douter(g, carry):
        for b in (0, 1):
            j = g * 2 + b
            pltpu.make_async_copy(
                ei1_hbm.at[pl.ds(gbase(j), CH)], idx[b], seml[b]).wait()
            pltpu.async_copy(ones_v, s_sh.at[idx[b]], sems[b], add=True)

            o = 1 - b

            @pl.when(j >= 1)
            def _():
                pltpu.make_async_copy(
                    ones_v, s_sh.at[idx[o]], sems[o]).wait()

            @pl.when(j + 1 < NFULL)
            def _():
                pltpu.async_copy(ei1_hbm.at[pl.ds(gbase(j + 1), CH)],
                                 idx[o], seml[o])

        return carry

    lax.fori_loop(0, NFULL // 2, douter, 0)
    pltpu.make_async_copy(ones_v, s_sh.at[idx[1]], sems[1]).wait()

    @pl.when(wid < NCH - NFULL * NW)
    def _():
        gb = (NFULL * NW + wid) * CH
        pltpu.sync_copy(ei1_hbm.at[pl.ds(gb, CH)], idx[0])
        pltpu.sync_copy(ones_v, s_sh.at[idx[0]], add=True)

    plsc.subcore_barrier()
    pltpu.sync_copy(s_sh.at[pl.ds(r0, ROWS_PER_SUB)],
                    dg_hbm.at[pl.ds(c * NPAD + r0, ROWS_PER_SUB)])


def _deg(ei1, zs, ones):
    f = pl.kernel(
        _deg_body,
        out_type=jax.ShapeDtypeStruct((NCORE * NPAD, D), jnp.float32),
        mesh=plsc.VectorSubcoreMesh(core_axis_name="c", subcore_axis_name="s"),
        scratch_types=[
            pltpu.VMEM((CH,), jnp.int32),
            pltpu.VMEM((CH,), jnp.int32),
            pltpu.VMEM((CH, D), jnp.float32),
            pltpu.VMEM_SHARED((NPAD, D), jnp.float32),
            pltpu.SemaphoreType.DMA,
            pltpu.SemaphoreType.DMA,
            pltpu.SemaphoreType.DMA,
            pltpu.SemaphoreType.DMA,
        ],
    )
    return f(ei1, zs, ones)


def _final_body(sp0_ref, sp1_ref, sp2_ref, dg_ref, s1_ref, s2_ref,
                bng_ref, bnb_ref, wo_ref, bo_ref, feats_ref, o_ref):
    s_sum = (sp0_ref[0] + sp0_ref[1] + sp1_ref[0] + sp1_ref[1]
             + sp2_ref[0] + sp2_ref[1])
    d_sum = dg_ref[0] + dg_ref[1]
    m = s1_ref[...] / float(E)
    v = s2_ref[...] / float(E) - m * m
    a = bng_ref[...] / jnp.sqrt(v + 1e-5)
    cv = bnb_ref[...] - m * a
    kv = jnp.dot(cv, wo_ref[...], preferred_element_type=jnp.float32)
    out = (jnp.dot(s_sum * a, wo_ref[...], preferred_element_type=jnp.float32)
           + d_sum * kv + bo_ref[...])
    out = _hyp(out)
    out = 1.0507009873554805 * jnp.where(
        out > 0, out, 1.6732632423543772 * (jnp.exp(jnp.minimum(out, 0.0)) - 1.0))
    o_ref[...] = out + feats_ref[...]


def _final(sps, dg, s1, s2, bn_g, bn_b, woT, bo, feats):
    vec = pl.BlockSpec((1, D), lambda i: (0, 0))
    acc = pl.BlockSpec((NCORE, BN_, D), lambda i: (0, i, 0))
    return pl.pallas_call(
        _final_body,
        grid=(N // BN_,),
        in_specs=[
            acc, acc, acc, acc,
            vec, vec, vec, vec,
            pl.BlockSpec((D, D), lambda i: (0, 0)),
            vec,
            pl.BlockSpec((BN_, D), lambda i: (i, 0)),
        ],
        out_specs=pl.BlockSpec((BN_, D), lambda i: (i, 0)),
        out_shape=jax.ShapeDtypeStruct((N, D), jnp.float32),
    )(sps[0], sps[1], sps[2], dg, s1, s2, bn_g, bn_b, woT, bo, feats)


def kernel(features, edge_index, c, W1, b1, ln_g, ln_b, W2, b2, bn_g, bn_b,
           Wo, bo):
    del c
    f32 = jnp.float32

    feats = _node_transform(features)

    ei0 = edge_index[0]
    ei1 = edge_index[1]
    gathered = [_make_gather(sl)(feats, ei0, ei1) for sl in SLICES]

    w1aT = W1[:, :D].T
    w1bT = W1[:, D:].T
    w2T = W2.T
    mlp = [_edge_mlp(sg, dg_, w1aT, w1bT, b1[None], ln_g[None], ln_b[None],
                     w2T, b2[None]) for sg, dg_ in gathered]

    zs = jnp.zeros((NPAD, D), f32)
    ones = jnp.ones((CH, D), f32)
    sps = [_make_scatter(sl)(h2, ei1, zs)
           for sl, (h2, _, _) in zip(SLICES, mlp)]
    dg = _deg(ei1, zs, ones)

    sps = [sp.reshape(NCORE, NPAD, D) for sp in sps]
    dg = dg.reshape(NCORE, NPAD, D)
    s1 = mlp[0][1] + mlp[1][1] + mlp[2][1]
    s2 = mlp[0][2] + mlp[1][2] + mlp[2][2]
    return _final(sps, dg, s1, s2, bn_g[None], bn_b[None], Wo.T, bo[None],
                  feats)


# speedup vs baseline: 5.3527x; 1.0467x over previous
import jax
import jax.numpy as jnp
from jax import lax
from jax.experimental import pallas as pl
from jax.experimental.pallas import tpu as pltpu
from jax.experimental.pallas import tpu_sc as plsc

N = 10000
E = 320000
D = 128

CH = 128
NCH = E // CH
NSUB = 16
NCORE = 2
NW = NCORE * NSUB
NPAD = 10240
ROWS_PER_SUB = NPAD // NSUB

BE = 2560
BN_ = 2000


def _norm(x):
    return jnp.maximum(jnp.sqrt(jnp.sum(x * x, axis=-1, keepdims=True)), 1e-15)


def _hyp(x):
    n = _norm(x)
    e = jnp.tanh(n) * x / n
    ne = _norm(e)
    maxn = 1.0 - 1e-05
    e = jnp.where(ne > maxn, e / ne * maxn, e)
    n3 = _norm(e)
    atanh = 0.5 * (jnp.log1p(n3) - jnp.log1p(-n3))
    return atanh / n3 * e


def _elu(x):
    return jnp.where(x > 0, x, jnp.exp(jnp.minimum(x, 0.0)) - 1.0)


def _node_body(x_ref, o_ref):
    o_ref[...] = _hyp(x_ref[...])


def _node_transform(features):
    blk = pl.BlockSpec((BN_, D), lambda i: (i, 0))
    return pl.pallas_call(
        _node_body,
        grid=(N // BN_,),
        in_specs=[blk],
        out_specs=blk,
        out_shape=jax.ShapeDtypeStruct((N, D), jnp.float32),
    )(features)


NFULL = NCH // NW

SLICES = ((0, 20, 0),
          (640, 30, 0),
          (1600, 28, 4))


def _make_gather_body(c0, nfull, nrem):
    def body(feats_hbm, ei0_hbm, ei1_hbm, srcg_hbm, dstg_hbm,
             idx0a, idx1a, idx0b, idx1b, r0a, r1a, r0b, r1b,
             semi_a, semi_b, semg_a, semg_b, semw_a, semw_b):
        c = lax.axis_index("c")
        s = lax.axis_index("s")
        wid = c * NSUB + s
        idx = ((idx0a, idx1a), (idx0b, idx1b))
        rows = ((r0a, r1a), (r0b, r1b))
        semi = (semi_a, semi_b)
        semg = (semg_a, semg_b)
        semw = (semw_a, semw_b)

        def gbase(j):
            return (c0 + wid + NW * j) * CH

        def lbase(j):
            return (wid + NW * j) * CH

        for b in (0, 1):
            pb = gbase(b)
            pltpu.async_copy(ei0_hbm.at[pl.ds(pb, CH)], idx[b][0], semi[b])
            pltpu.async_copy(ei1_hbm.at[pl.ds(pb, CH)], idx[b][1], semi[b])

        def outer(g, carry):
            for b in (0, 1):
                j = g * 2 + b
                base = gbase(j)
                lb = lbase(j)

                @pl.when(j >= 2)
                def _():
                    pv = lbase(j - 2)
                    pltpu.make_async_copy(
                        rows[b][0], srcg_hbm.at[pl.ds(pv, CH)],
                        semw[b]).wait()
                    pltpu.make_async_copy(
                        rows[b][1], dstg_hbm.at[pl.ds(pv, CH)],
                        semw[b]).wait()

                pltpu.make_async_copy(
                    ei0_hbm.at[pl.ds(base, CH)], idx[b][0], semi[b]).wait()
                pltpu.make_async_copy(
                    ei1_hbm.at[pl.ds(base, CH)], idx[b][1], semi[b]).wait()
                cp0 = pltpu.async_copy(feats_hbm.at[idx[b][0]], rows[b][0],
                                       semg[b])
                cp1 = pltpu.async_copy(feats_hbm.at[idx[b][1]], rows[b][1],
                                       semg[b])
                cp0.wait()
                cp1.wait()
                pltpu.async_copy(rows[b][0], srcg_hbm.at[pl.ds(lb, CH)],
                                 semw[b])
                pltpu.async_copy(rows[b][1], dstg_hbm.at[pl.ds(lb, CH)],
                                 semw[b])

                @pl.when(j + 2 < nfull)
                def _():
                    nb = gbase(j + 2)
                    pltpu.async_copy(ei0_hbm.at[pl.ds(nb, CH)], idx[b][0],
                                     semi[b])
                    pltpu.async_copy(ei1_hbm.at[pl.ds(nb, CH)], idx[b][1],
                                     semi[b])

            return carry

        lax.fori_loop(0, nfull // 2, outer, 0)

        for b in (0, 1):
            pv = lbase(nfull - 2 + b)
            pltpu.make_async_copy(rows[b][0], srcg_hbm.at[pl.ds(pv, CH)],
                                  semw[b]).wait()
            pltpu.make_async_copy(rows[b][1], dstg_hbm.at[pl.ds(pv, CH)],
                                  semw[b]).wait()

        if nrem:
            @pl.when(wid < nrem)
            def _():
                base = (c0 + nfull * NW + wid) * CH
                lb = (nfull * NW + wid) * CH
                pltpu.sync_copy(ei0_hbm.at[pl.ds(base, CH)], idx[0][0])
                pltpu.sync_copy(ei1_hbm.at[pl.ds(base, CH)], idx[0][1])
                cp0 = pltpu.async_copy(feats_hbm.at[idx[0][0]], rows[0][0],
                                       semg[0])
                cp1 = pltpu.async_copy(feats_hbm.at[idx[0][1]], rows[0][1],
                                       semg[0])
                cp0.wait()
                cp1.wait()
                pltpu.sync_copy(rows[0][0], srcg_hbm.at[pl.ds(lb, CH)])
                pltpu.sync_copy(rows[0][1], dstg_hbm.at[pl.ds(lb, CH)])

    return body


def _make_gather(sl):
    c0, nfull, nrem = sl
    es = (nfull * NW + nrem) * CH
    return pl.kernel(
        _make_gather_body(c0, nfull, nrem),
        out_type=(jax.ShapeDtypeStruct((es, D), jnp.float32),
                  jax.ShapeDtypeStruct((es, D), jnp.float32)),
        mesh=plsc.VectorSubcoreMesh(core_axis_name="c", subcore_axis_name="s"),
        scratch_types=[
            pltpu.VMEM((CH,), jnp.int32),
            pltpu.VMEM((CH,), jnp.int32),
            pltpu.VMEM((CH,), jnp.int32),
            pltpu.VMEM((CH,), jnp.int32),
            pltpu.VMEM((CH, D), jnp.float32),
            pltpu.VMEM((CH, D), jnp.float32),
            pltpu.VMEM((CH, D), jnp.float32),
            pltpu.VMEM((CH, D), jnp.float32),
            pltpu.SemaphoreType.DMA,
            pltpu.SemaphoreType.DMA,
            pltpu.SemaphoreType.DMA,
            pltpu.SemaphoreType.DMA,
            pltpu.SemaphoreType.DMA,
            pltpu.SemaphoreType.DMA,
        ],
    )


def _edge_body(src_ref, dst_ref, w1a_ref, w1b_ref, b1_ref, lng_ref, lnb_ref,
               w2_ref, b2_ref, h2_ref, s1_ref, s2_ref):
    src = src_ref[...].astype(jnp.float32)
    dst = dst_ref[...].astype(jnp.float32)
    multi = jnp.sum(src * dst, axis=-1, keepdims=True)
    dd = src - dst
    dist = jnp.sqrt(jnp.sum(dd * dd, axis=-1, keepdims=True))
    z = 2.0 * dist + 2.0 * (dist * dist * dist / 3.0 + multi * dist * dist)
    coef = 1.0 - jax.nn.sigmoid(z)
    h = (jnp.dot((1.0 + coef) * src, w1a_ref[...],
                 preferred_element_type=jnp.float32)
         + jnp.dot(dst, w1b_ref[...], preferred_element_type=jnp.float32)
         + b1_ref[...])
    h = _elu(h)
    mu = jnp.mean(h, axis=-1, keepdims=True)
    hc = h - mu
    var = jnp.mean(hc * hc, axis=-1, keepdims=True)
    h = hc / jnp.sqrt(var + 1e-5) * lng_ref[...] + lnb_ref[...]
    h = jnp.dot(h, w2_ref[...], preferred_element_type=jnp.float32) + b2_ref[...]
    h = _elu(h)
    h2_ref[...] = h

    @pl.when(pl.program_id(0) == 0)
    def _():
        s1_ref[...] = jnp.zeros_like(s1_ref)
        s2_ref[...] = jnp.zeros_like(s2_ref)

    s1_ref[...] += jnp.sum(h, axis=0, keepdims=True)
    s2_ref[...] += jnp.sum(h * h, axis=0, keepdims=True)


def _edge_mlp(srcg, dstg, w1aT, w1bT, b1, ln_g, ln_b, w2T, b2):
    full = pl.BlockSpec((D, D), lambda i: (0, 0))
    vec = pl.BlockSpec((1, D), lambda i: (0, 0))
    return pl.pallas_call(
        _edge_body,
        grid=(srcg.shape[0] // BE,),
        in_specs=[
            pl.BlockSpec((BE, D), lambda i: (i, 0)),
            pl.BlockSpec((BE, D), lambda i: (i, 0)),
            full, full, vec, vec, vec, full, vec,
        ],
        out_specs=[
            pl.BlockSpec((BE, D), lambda i: (i, 0)),
            vec, vec,
        ],
        out_shape=[
            jax.ShapeDtypeStruct((srcg.shape[0], D), jnp.float32),
            jax.ShapeDtypeStruct((1, D), jnp.float32),
            jax.ShapeDtypeStruct((1, D), jnp.float32),
        ],
    )(srcg, dstg, w1aT, w1bT, b1, ln_g, ln_b, w2T, b2)


def _make_scatter_body(c0, nfull, nrem):
    def body(h2_hbm, ei1_hbm, zs_hbm, sp_hbm,
             idxa, idxb, h2a, h2b, s_sh,
             seml_a, seml_b, sems_a, sems_b):
        c = lax.axis_index("c")
        s = lax.axis_index("s")
        wid = c * NSUB + s
        r0 = s * ROWS_PER_SUB
        idx = (idxa, idxb)
        h2v = (h2a, h2b)
        seml = (seml_a, seml_b)
        sems = (sems_a, sems_b)

        @pl.when(s == 0)
        def _():
            pltpu.sync_copy(zs_hbm, s_sh)

        plsc.subcore_barrier()

        def gbase(j):
            return (c0 + wid + NW * j) * CH

        def lbase(j):
            return (wid + NW * j) * CH

        pltpu.async_copy(ei1_hbm.at[pl.ds(gbase(0), CH)], idx[0], seml[0])
        pltpu.async_copy(h2_hbm.at[pl.ds(lbase(0), CH)], h2v[0], seml[0])

        def outer(g, carry):
            for b in (0, 1):
                j = g * 2 + b
                pltpu.make_async_copy(
                    ei1_hbm.at[pl.ds(gbase(j), CH)], idx[b], seml[b]).wait()
                pltpu.make_async_copy(
                    h2_hbm.at[pl.ds(lbase(j), CH)], h2v[b], seml[b]).wait()
                pltpu.async_copy(h2v[b], s_sh.at[idx[b]], sems[b], add=True)

                o = 1 - b

                @pl.when(j >= 1)
                def _():
                    pltpu.make_async_copy(
                        h2v[o], s_sh.at[idx[o]], sems[o]).wait()

                @pl.when(j + 1 < nfull)
                def _():
                    pltpu.async_copy(ei1_hbm.at[pl.ds(gbase(j + 1), CH)],
                                     idx[o], seml[o])
                    pltpu.async_copy(h2_hbm.at[pl.ds(lbase(j + 1), CH)],
                                     h2v[o], seml[o])

            return carry

        lax.fori_loop(0, nfull // 2, outer, 0)
        pltpu.make_async_copy(h2v[1], s_sh.at[idx[1]], sems[1]).wait()

        if nrem:
            @pl.when(wid < nrem)
            def _():
                gb = (c0 + nfull * NW + wid) * CH
                lb = (nfull * NW + wid) * CH
                pltpu.sync_copy(ei1_hbm.at[pl.ds(gb, CH)], idx[0])
                pltpu.sync_copy(h2_hbm.at[pl.ds(lb, CH)], h2v[0])
                pltpu.sync_copy(h2v[0], s_sh.at[idx[0]], add=True)

        plsc.subcore_barrier()
        pltpu.sync_copy(s_sh.at[pl.ds(r0, ROWS_PER_SUB)],
                        sp_hbm.at[pl.ds(c * NPAD + r0, ROWS_PER_SUB)])

    return body


def _make_scatter(sl):
    c0, nfull, nrem = sl
    return pl.kernel(
        _make_scatter_body(c0, nfull, nrem),
        out_type=jax.ShapeDtypeStruct((NCORE * NPAD, D), jnp.float32),
        mesh=plsc.VectorSubcoreMesh(core_axis_name="c", subcore_axis_name="s"),
        scratch_types=[
            pltpu.VMEM((CH,), jnp.int32),
            pltpu.VMEM((CH,), jnp.int32),
            pltpu.VMEM((CH, D), jnp.float32),
            pltpu.VMEM((CH, D), jnp.float32),
            pltpu.VMEM_SHARED((NPAD, D), jnp.float32),
            pltpu.SemaphoreType.DMA,
            pltpu.SemaphoreType.DMA,
            pltpu.SemaphoreType.DMA,
            pltpu.SemaphoreType.DMA,
        ],
    )


def _deg_body(ei1_hbm, zs_hbm, ones_hbm, dg_hbm,
              idxa, idxb, ones_v, s_sh, seml_a, seml_b, sems_a, sems_b):
    c = lax.axis_index("c")
    s = lax.axis_index("s")
    wid = c * NSUB + s
    r0 = s * ROWS_PER_SUB
    idx = (idxa, idxb)
    seml = (seml_a, seml_b)
    sems = (sems_a, sems_b)

    @pl.when(s == 0)
    def _():
        pltpu.sync_copy(zs_hbm, s_sh)

    pltpu.sync_copy(ones_hbm, ones_v)
    plsc.subcore_barrier()

    def gbase(j):
        return (wid + NW * j) * CH

    pltpu.async_copy(ei1_hbm.at[pl.ds(gbase(0), CH)], idx[0], seml[0])

    def douter(g, carry):
        for b in (0, 1):
            j = g * 2 + b
            pltpu.make_async_copy(
                ei1_hbm.at[pl.ds(gbase(j), CH)], idx[b], seml[b]).wait()
            pltpu.async_copy(ones_v, s_sh.at[idx[b]], sems[b], add=True)

            o = 1 - b

            @pl.when(j >= 1)
            def _():
                pltpu.make_async_copy(
                    ones_v, s_sh.at[idx[o]], sems[o]).wait()

            @pl.when(j + 1 < NFULL)
            def _():
                pltpu.async_copy(ei1_hbm.at[pl.ds(gbase(j + 1), CH)],
                                 idx[o], seml[o])

        return carry

    lax.fori_loop(0, NFULL // 2, douter, 0)
    pltpu.make_async_copy(ones_v, s_sh.at[idx[1]], sems[1]).wait()

    @pl.when(wid < NCH - NFULL * NW)
    def _():
        gb = (NFULL * NW + wid) * CH
        pltpu.sync_copy(ei1_hbm.at[pl.ds(gb, CH)], idx[0])
        pltpu.sync_copy(ones_v, s_sh.at[idx[0]], add=True)

    plsc.subcore_barrier()
    pltpu.sync_copy(s_sh.at[pl.ds(r0, ROWS_PER_SUB)],
                    dg_hbm.at[pl.ds(c * NPAD + r0, ROWS_PER_SUB)])


def _deg(ei1, zs, ones):
    f = pl.kernel(
        _deg_body,
        out_type=jax.ShapeDtypeStruct((NCORE * NPAD, D), jnp.float32),
        mesh=plsc.VectorSubcoreMesh(core_axis_name="c", subcore_axis_name="s"),
        scratch_types=[
            pltpu.VMEM((CH,), jnp.int32),
            pltpu.VMEM((CH,), jnp.int32),
            pltpu.VMEM((CH, D), jnp.float32),
            pltpu.VMEM_SHARED((NPAD, D), jnp.float32),
            pltpu.SemaphoreType.DMA,
            pltpu.SemaphoreType.DMA,
            pltpu.SemaphoreType.DMA,
            pltpu.SemaphoreType.DMA,
        ],
    )
    return f(ei1, zs, ones)


def _final_body(sp0_ref, sp1_ref, sp2_ref, dg_ref, s1_ref, s2_ref,
                bng_ref, bnb_ref, wo_ref, bo_ref, feats_ref, o_ref):
    s_sum = (sp0_ref[0] + sp0_ref[1] + sp1_ref[0] + sp1_ref[1]
             + sp2_ref[0] + sp2_ref[1])
    d_sum = dg_ref[0] + dg_ref[1]
    m = s1_ref[...] / float(E)
    v = s2_ref[...] / float(E) - m * m
    a = bng_ref[...] / jnp.sqrt(v + 1e-5)
    cv = bnb_ref[...] - m * a
    kv = jnp.dot(cv, wo_ref[...], preferred_element_type=jnp.float32)
    out = (jnp.dot(s_sum * a, wo_ref[...], preferred_element_type=jnp.float32)
           + d_sum * kv + bo_ref[...])
    out = _hyp(out)
    out = 1.0507009873554805 * jnp.where(
        out > 0, out, 1.6732632423543772 * (jnp.exp(jnp.minimum(out, 0.0)) - 1.0))
    o_ref[...] = out + feats_ref[...]


def _final(sps, dg, s1, s2, bn_g, bn_b, woT, bo, feats):
    vec = pl.BlockSpec((1, D), lambda i: (0, 0))
    acc = pl.BlockSpec((NCORE, BN_, D), lambda i: (0, i, 0))
    return pl.pallas_call(
        _final_body,
        grid=(N // BN_,),
        in_specs=[
            acc, acc, acc, acc,
            vec, vec, vec, vec,
            pl.BlockSpec((D, D), lambda i: (0, 0)),
            vec,
            pl.BlockSpec((BN_, D), lambda i: (i, 0)),
        ],
        out_specs=pl.BlockSpec((BN_, D), lambda i: (i, 0)),
        out_shape=jax.ShapeDtypeStruct((N, D), jnp.float32),
    )(sps[0], sps[1], sps[2], dg, s1, s2, bn_g, bn_b, woT, bo, feats)


def kernel(features, edge_index, c, W1, b1, ln_g, ln_b, W2, b2, bn_g, bn_b,
           Wo, bo):
    del c
    f32 = jnp.float32

    feats = _node_transform(features)

    ei0 = edge_index[0]
    ei1 = edge_index[1]
    gathered = [_make_gather(sl)(feats, ei0, ei1) for sl in SLICES]

    w1aT = W1[:, :D].T
    w1bT = W1[:, D:].T
    w2T = W2.T
    mlp = [_edge_mlp(sg, dg_, w1aT, w1bT, b1[None], ln_g[None], ln_b[None],
                     w2T, b2[None]) for sg, dg_ in gathered]

    zs = jnp.zeros((NPAD, D), f32)
    ones = jnp.ones((CH, D), f32)
    sp0 = _make_scatter(SLICES[0])(mlp[0][0], ei1, zs)
    dg = _deg(ei1, zs, ones)
    sp1 = _make_scatter(SLICES[1])(mlp[1][0], ei1, zs)
    sp2 = _make_scatter(SLICES[2])(mlp[2][0], ei1, zs)
    sps = [sp0, sp1, sp2]

    sps = [sp.reshape(NCORE, NPAD, D) for sp in sps]
    dg = dg.reshape(NCORE, NPAD, D)
    s1 = mlp[0][1] + mlp[1][1] + mlp[2][1]
    s2 = mlp[0][2] + mlp[1][2] + mlp[2][2]
    return _final(sps, dg, s1, s2, bn_g[None], bn_b[None], Wo.T, bo[None],
                  feats)
